# baseline (speedup 1.0000x reference)
import jax
import jax.numpy as jnp
from jax import lax
from jax.experimental import pallas as pl
from jax.experimental.pallas import tpu as pltpu
from jax.experimental.pallas import tpu_sc as plsc

B, S, D = 128, 197, 768
P, L, K = 512, 16, 8
BBLK = 16

NC, NS, LANES = 2, 16, 16
NW = NC * NS
DPW = D // NW
NV = P // LANES


def _mean_norm_kernel(x_ref, pk_ref, xn_ref, x2_ref, sgn_ref, w3_ref,
                      pn_ref, pnT_ref):
    i = pl.program_id(0)

    @pl.when(i == 0)
    def _():
        pk = pk_ref[...]
        pss = jnp.sum(pk * pk, axis=1, keepdims=True)
        pnorm = pk * jax.lax.rsqrt(jnp.maximum(pss, 1e-12))
        t = pnorm.T
        pnT_ref[...] = t
        pn_ref[...] = t.reshape(D, 1, P)

    xm = jnp.sum(x_ref[...], axis=1) * (1.0 / S)
    ss = jnp.sum(xm * xm, axis=1, keepdims=True)
    xn = xm * jax.lax.rsqrt(jnp.maximum(ss, 1e-12))
    x2_ref[...] = xn
    xn_ref[...] = xn.T.reshape(D, BBLK, 1)
    pos = jnp.sum((xn > 0).astype(jnp.int32), axis=0, keepdims=True)
    neg = jnp.sum((xn < 0).astype(jnp.int32), axis=0, keepdims=True)
    cur = jnp.concatenate([pos, neg], axis=0)

    @pl.when(i == 0)
    def _():
        sgn_ref[...] = cur

    @pl.when(i != 0)
    def _():
        sgn_ref[...] = sgn_ref[...] + cur

    @pl.when(i == B // BBLK - 1)
    def _():
        s = sgn_ref[...]
        w3_ref[...] = jnp.broadcast_to(s[:, :, None], (2, D, LANES))


SIMB = 16


def _sim_kernel(x2_ref, pnT_ref, sim_ref):
    xt = x2_ref[...].T
    pnT = pnT_ref[...]
    for j in range(SIMB):
        sim_ref[j] = xt[:, j:j + 1] * pnT


def _tree(op, vs):
    while len(vs) > 1:
        nxt = [op(vs[i], vs[i + 1]) for i in range(0, len(vs) - 1, 2)]
        if len(vs) % 2:
            nxt.append(vs[-1])
        vs = nxt
    return vs[0]


def _sc_select_body(pnT_hbm, w3_hbm, out_hbm, pn_v, wp_v, wn_v, counts_v):
    wid = lax.axis_index("s") * NC + lax.axis_index("c")
    base = wid * DPW
    pltpu.sync_copy(pnT_hbm.at[pl.ds(base, DPW)], pn_v)
    pltpu.sync_copy(w3_hbm.at[0, pl.ds(base, DPW)], wp_v)
    pltpu.sync_copy(w3_hbm.at[1, pl.ds(base, DPW)], wn_v)

    lane = lax.iota(jnp.int32, LANES)
    zero16 = jnp.zeros((LANES,), jnp.int32)
    for j in range(NV):
        counts_v[pl.ds(j * LANES, LANES)] = zero16

    def col_body(r, carry):
        idxs = [lane + j * LANES for j in range(NV)]

        def side(vals, wvec):
            a = list(vals)
            sel = zero16
            for k in range(K):
                m = jnp.max(_tree(jnp.maximum, a))
                cand = [jnp.where(a[j] == m, idxs[j], P) for j in range(NV)]
                pstar = jnp.min(_tree(jnp.minimum, cand))
                sel = jnp.where(lane == k, pstar, sel)
                a = [jnp.where(idxs[j] == pstar, -jnp.inf, a[j])
                     for j in range(NV)]
            plsc.addupdate_scatter(counts_v, [sel], wvec, mask=lane < K)

        vals = [pn_v[r, pl.ds(j * LANES, LANES)] for j in range(NV)]
        side(vals, wp_v[r, :])
        side([-v for v in vals], wn_v[r, :])
        return carry

    lax.fori_loop(0, DPW, col_body, 0)
    pltpu.sync_copy(counts_v, out_hbm.at[wid])


def _merge_kernel(partials_ref, sgn_ref, major_ref):
    counts = jnp.sum(partials_ref[...], axis=0, keepdims=True)
    nzero = B * D - jnp.sum(sgn_ref[...])
    pidx = jax.lax.broadcasted_iota(jnp.int32, (1, P), 1)
    counts = counts + jnp.where(pidx < K, nzero, 0)

    kidx = jax.lax.broadcasted_iota(jnp.int32, (1, K), 1)
    major = jnp.zeros((1, K), jnp.int32)
    for k in range(K):
        m = jnp.max(counts, axis=1, keepdims=True)
        cand = jnp.where(counts == m, pidx, P)
        amin = jnp.min(cand, axis=1, keepdims=True)
        major = jnp.where(kidx == k, amin, major)
        counts = jnp.where(pidx == amin, -1, counts)
    major_ref[...] = major


GBLK = 128


def _gather_kernel(major_sref, prompt_ref, bp_ref, idx_ref):
    k = pl.program_id(0)
    bp_ref[...] = jnp.broadcast_to(prompt_ref[...], (GBLK, L, D))
    idx_ref[...] = jnp.full((GBLK, 1, L, D), major_sref[k], jnp.int32)


def kernel(x_embed, prompt, prompt_key):
    xn, x2, sgn, w3, pn, pnT = pl.pallas_call(
        _mean_norm_kernel,
        grid=(B // BBLK,),
        in_specs=[pl.BlockSpec((BBLK, S, D), lambda i: (i, 0, 0)),
                  pl.BlockSpec((P, D), lambda i: (0, 0))],
        out_specs=[pl.BlockSpec((D, BBLK, 1), lambda i: (0, i, 0)),
                   pl.BlockSpec((BBLK, D), lambda i: (i, 0)),
                   pl.BlockSpec((2, D), lambda i: (0, 0)),
                   pl.BlockSpec((2, D, LANES), lambda i: (0, 0, 0)),
                   pl.BlockSpec((D, 1, P), lambda i: (0, 0, 0)),
                   pl.BlockSpec((D, P), lambda i: (0, 0))],
        out_shape=[jax.ShapeDtypeStruct((D, B, 1), jnp.float32),
                   jax.ShapeDtypeStruct((B, D), jnp.float32),
                   jax.ShapeDtypeStruct((2, D), jnp.int32),
                   jax.ShapeDtypeStruct((2, D, LANES), jnp.int32),
                   jax.ShapeDtypeStruct((D, 1, P), jnp.float32),
                   jax.ShapeDtypeStruct((D, P), jnp.float32)],
    )(x_embed, prompt_key)

    sc_select = pl.kernel(
        _sc_select_body,
        mesh=plsc.VectorSubcoreMesh(core_axis_name="c", subcore_axis_name="s"),
        compiler_params=pltpu.CompilerParams(needs_layout_passes=False),
        out_type=jax.ShapeDtypeStruct((NW, P), jnp.int32),
        scratch_types=[pltpu.VMEM((DPW, P), jnp.float32),
                       pltpu.VMEM((DPW, LANES), jnp.int32),
                       pltpu.VMEM((DPW, LANES), jnp.int32),
                       pltpu.VMEM((P,), jnp.int32)],
    )
    partials = sc_select(pnT, w3)

    similarity = pl.pallas_call(
        _sim_kernel,
        grid=(B // SIMB,),
        in_specs=[pl.BlockSpec((SIMB, D), lambda b: (b, 0)),
                  pl.BlockSpec((D, P), lambda b: (0, 0))],
        out_specs=pl.BlockSpec((SIMB, D, P), lambda b: (b, 0, 0)),
        out_shape=jax.ShapeDtypeStruct((B, D, P), jnp.float32),
    )(x2, pnT)

    major = pl.pallas_call(
        _merge_kernel,
        in_specs=[pl.BlockSpec((NW, P), lambda: (0, 0)),
                  pl.BlockSpec((2, D), lambda: (0, 0))],
        out_specs=pl.BlockSpec((1, K), lambda: (0, 0)),
        out_shape=jax.ShapeDtypeStruct((1, K), jnp.int32),
    )(partials, sgn)

    batched_prompt, idx4 = pl.pallas_call(
        _gather_kernel,
        grid_spec=pltpu.PrefetchScalarGridSpec(
            num_scalar_prefetch=1,
            grid=(K, B // GBLK),
            in_specs=[pl.BlockSpec((1, L, D), lambda k, b, m: (m[k], 0, 0))],
            out_specs=[pl.BlockSpec((GBLK, L, D), lambda k, b, m: (b, k, 0)),
                       pl.BlockSpec((GBLK, 1, L, D),
                                    lambda k, b, m: (b, k, 0, 0))],
        ),
        out_shape=[jax.ShapeDtypeStruct((B, K * L, D), jnp.float32),
                   jax.ShapeDtypeStruct((B, K, L, D), jnp.int32)],
    )(major.reshape(K), prompt)

    return (batched_prompt, similarity, xn, pn, idx4)


# speedup vs baseline: 1.0071x; 1.0071x over previous
import jax
import jax.numpy as jnp
from jax import lax
from jax.experimental import pallas as pl
from jax.experimental.pallas import tpu as pltpu
from jax.experimental.pallas import tpu_sc as plsc

B, S, D = 128, 197, 768
P, L, K = 512, 16, 8
BBLK = 16

NC, NS, LANES = 2, 16, 16
NW = NC * NS
DPW = D // NW
NV = P // LANES


def _mean_norm_kernel(x_ref, pk_ref, xn_ref, x2_ref, sgn_ref, w3_ref,
                      pn_ref, pnT_ref):
    i = pl.program_id(0)

    @pl.when(i == 0)
    def _():
        pk = pk_ref[...]
        pss = jnp.sum(pk * pk, axis=1, keepdims=True)
        pnorm = pk * jax.lax.rsqrt(jnp.maximum(pss, 1e-12))
        t = pnorm.T
        pnT_ref[...] = t
        pn_ref[...] = t.reshape(D, 1, P)

    xm = jnp.sum(x_ref[...], axis=1) * (1.0 / S)
    ss = jnp.sum(xm * xm, axis=1, keepdims=True)
    xn = xm * jax.lax.rsqrt(jnp.maximum(ss, 1e-12))
    x2_ref[...] = xn
    xn_ref[...] = xn.T.reshape(D, BBLK, 1)
    pos = jnp.sum((xn > 0).astype(jnp.int32), axis=0, keepdims=True)
    neg = jnp.sum((xn < 0).astype(jnp.int32), axis=0, keepdims=True)
    cur = jnp.concatenate([pos, neg], axis=0)

    @pl.when(i == 0)
    def _():
        sgn_ref[...] = cur

    @pl.when(i != 0)
    def _():
        sgn_ref[...] = sgn_ref[...] + cur

    @pl.when(i == B // BBLK - 1)
    def _():
        s = sgn_ref[...]
        w3_ref[...] = jnp.broadcast_to(s[:, :, None], (2, D, LANES))


SIMB = 16


def _sim_kernel(x2_ref, pnT_ref, sim_ref):
    xt = x2_ref[...].T
    pnT = pnT_ref[...]
    for j in range(SIMB):
        sim_ref[j] = xt[:, j:j + 1] * pnT


def _tree(op, vs):
    while len(vs) > 1:
        nxt = [op(vs[i], vs[i + 1]) for i in range(0, len(vs) - 1, 2)]
        if len(vs) % 2:
            nxt.append(vs[-1])
        vs = nxt
    return vs[0]


def _sc_select_body(pnT_hbm, w3_hbm, out_hbm, pn_v, wp_v, wn_v, counts_v):
    wid = lax.axis_index("s") * NC + lax.axis_index("c")
    base = wid * DPW
    pltpu.sync_copy(pnT_hbm.at[pl.ds(base, DPW)], pn_v)
    pltpu.sync_copy(w3_hbm.at[0, pl.ds(base, DPW)], wp_v)
    pltpu.sync_copy(w3_hbm.at[1, pl.ds(base, DPW)], wn_v)

    lane = lax.iota(jnp.int32, LANES)
    zero16 = jnp.zeros((LANES,), jnp.int32)
    for j in range(NV):
        counts_v[pl.ds(j * LANES, LANES)] = zero16

    def col_body(r, carry):
        idxs = [lane + j * LANES for j in range(NV)]

        def side(vals, wvec):
            a = list(vals)
            sel = zero16
            for k in range(K):
                m = jnp.max(_tree(jnp.maximum, a))
                cand = [jnp.where(a[j] == m, idxs[j], P) for j in range(NV)]
                pstar = jnp.min(_tree(jnp.minimum, cand))
                sel = jnp.where(lane == k, pstar, sel)
                a = [jnp.where(idxs[j] == pstar, -jnp.inf, a[j])
                     for j in range(NV)]
            plsc.addupdate_scatter(counts_v, [sel], wvec, mask=lane < K)

        vals = [pn_v[r, pl.ds(j * LANES, LANES)] for j in range(NV)]
        side(vals, wp_v[r, :])
        side([-v for v in vals], wn_v[r, :])
        return carry

    lax.fori_loop(0, DPW, col_body, 0)
    pltpu.sync_copy(counts_v, out_hbm.at[wid])


def _merge_kernel(partials_ref, sgn_ref, major_ref):
    counts = jnp.sum(partials_ref[...], axis=0, keepdims=True)
    nzero = B * D - jnp.sum(sgn_ref[...])
    pidx = jax.lax.broadcasted_iota(jnp.int32, (1, P), 1)
    counts = counts + jnp.where(pidx < K, nzero, 0)

    kidx = jax.lax.broadcasted_iota(jnp.int32, (1, K), 1)
    major = jnp.zeros((1, K), jnp.int32)
    for k in range(K):
        m = jnp.max(counts, axis=1, keepdims=True)
        cand = jnp.where(counts == m, pidx, P)
        amin = jnp.min(cand, axis=1, keepdims=True)
        major = jnp.where(kidx == k, amin, major)
        counts = jnp.where(pidx == amin, -1, counts)
    major_ref[...] = major


GBLK = 64


def _gather_kernel(major_sref, prompt_ref, bp_ref, idx_ref):
    k = pl.program_id(0)
    bp_ref[...] = jnp.broadcast_to(prompt_ref[...], (GBLK, L, D))
    idx_ref[...] = jnp.full((GBLK, 1, L, D), major_sref[k], jnp.int32)


def kernel(x_embed, prompt, prompt_key):
    xn, x2, sgn, w3, pn, pnT = pl.pallas_call(
        _mean_norm_kernel,
        grid=(B // BBLK,),
        in_specs=[pl.BlockSpec((BBLK, S, D), lambda i: (i, 0, 0)),
                  pl.BlockSpec((P, D), lambda i: (0, 0))],
        out_specs=[pl.BlockSpec((D, BBLK, 1), lambda i: (0, i, 0)),
                   pl.BlockSpec((BBLK, D), lambda i: (i, 0)),
                   pl.BlockSpec((2, D), lambda i: (0, 0)),
                   pl.BlockSpec((2, D, LANES), lambda i: (0, 0, 0)),
                   pl.BlockSpec((D, 1, P), lambda i: (0, 0, 0)),
                   pl.BlockSpec((D, P), lambda i: (0, 0))],
        out_shape=[jax.ShapeDtypeStruct((D, B, 1), jnp.float32),
                   jax.ShapeDtypeStruct((B, D), jnp.float32),
                   jax.ShapeDtypeStruct((2, D), jnp.int32),
                   jax.ShapeDtypeStruct((2, D, LANES), jnp.int32),
                   jax.ShapeDtypeStruct((D, 1, P), jnp.float32),
                   jax.ShapeDtypeStruct((D, P), jnp.float32)],
    )(x_embed, prompt_key)

    sc_select = pl.kernel(
        _sc_select_body,
        mesh=plsc.VectorSubcoreMesh(core_axis_name="c", subcore_axis_name="s"),
        compiler_params=pltpu.CompilerParams(needs_layout_passes=False),
        out_type=jax.ShapeDtypeStruct((NW, P), jnp.int32),
        scratch_types=[pltpu.VMEM((DPW, P), jnp.float32),
                       pltpu.VMEM((DPW, LANES), jnp.int32),
                       pltpu.VMEM((DPW, LANES), jnp.int32),
                       pltpu.VMEM((P,), jnp.int32)],
    )
    partials = sc_select(pnT, w3)

    similarity = pl.pallas_call(
        _sim_kernel,
        grid=(B // SIMB,),
        in_specs=[pl.BlockSpec((SIMB, D), lambda b: (b, 0)),
                  pl.BlockSpec((D, P), lambda b: (0, 0))],
        out_specs=pl.BlockSpec((SIMB, D, P), lambda b: (b, 0, 0)),
        out_shape=jax.ShapeDtypeStruct((B, D, P), jnp.float32),
    )(x2, pnT)

    major = pl.pallas_call(
        _merge_kernel,
        in_specs=[pl.BlockSpec((NW, P), lambda: (0, 0)),
                  pl.BlockSpec((2, D), lambda: (0, 0))],
        out_specs=pl.BlockSpec((1, K), lambda: (0, 0)),
        out_shape=jax.ShapeDtypeStruct((1, K), jnp.int32),
    )(partials, sgn)

    batched_prompt, idx4 = pl.pallas_call(
        _gather_kernel,
        grid_spec=pltpu.PrefetchScalarGridSpec(
            num_scalar_prefetch=1,
            grid=(K, B // GBLK),
            in_specs=[pl.BlockSpec((1, L, D), lambda k, b, m: (m[k], 0, 0))],
            out_specs=[pl.BlockSpec((GBLK, L, D), lambda k, b, m: (b, k, 0)),
                       pl.BlockSpec((GBLK, 1, L, D),
                                    lambda k, b, m: (b, k, 0, 0))],
        ),
        out_shape=[jax.ShapeDtypeStruct((B, K * L, D), jnp.float32),
                   jax.ShapeDtypeStruct((B, K, L, D), jnp.int32)],
    )(major.reshape(K), prompt)

    return (batched_prompt, similarity, xn, pn, idx4)
